# baseline (speedup 1.0000x reference)
import functools

import jax
import jax.numpy as jnp
from jax import lax
from jax.experimental import pallas as pl
from jax.experimental.pallas import tpu as pltpu
from jax.experimental.pallas import tpu_sc as plsc

_VOCAB = 100000
_EMBED = 64
_HIDDEN = 128
_BATCH = 1024
_CTX = 20

_NC = 2
_NS = 16
_NW = _NC * _NS
_NIDX = _BATCH * _CTX
_B_PER_W = _NIDX // _NW
_GROW = 2 * _EMBED

_BB = 128
_VT = 1024
_NV = (_VOCAB + _VT - 1) // _VT


def _sc_gather_kernel(table_hbm, idx_hbm, out_hbm, idx_v, rows_v, sem):
    wid = lax.axis_index("s") * _NC + lax.axis_index("c")
    base = wid * _B_PER_W
    pltpu.sync_copy(idx_hbm.at[pl.ds(base, _B_PER_W)], idx_v)
    pltpu.async_copy(table_hbm.at[idx_v], rows_v, sem).wait()
    pltpu.sync_copy(rows_v, out_hbm.at[pl.ds(base, _B_PER_W)])


def _sc_gather(emb_table, flat_idx):
    mesh = plsc.VectorSubcoreMesh(core_axis_name="c", subcore_axis_name="s")
    k = functools.partial(
        pl.kernel,
        mesh=mesh,
        out_type=jax.ShapeDtypeStruct((_NIDX, _GROW), jnp.float32),
        scratch_types=[
            pltpu.VMEM((_B_PER_W,), jnp.int32),
            pltpu.VMEM((_B_PER_W, _GROW), jnp.float32),
            pltpu.SemaphoreType.DMA,
        ],
    )(_sc_gather_kernel)
    return k(emb_table, flat_idx)


def _pool_body(gath_ref, par_ref, wh_ref, bh_ref, hid_ref):
    acc = jnp.zeros((_BB, _EMBED), jnp.float32)
    for c in range(_CTX):
        g0 = gath_ref[:, c, :_EMBED]
        g1 = gath_ref[:, c, _EMBED:]
        s = par_ref[:, c:c + 1]
        acc = acc + g0 * (1.0 - s) + g1 * s
    pooled = acc * (1.0 / _CTX)
    h = pooled @ wh_ref[...] + bh_ref[...]
    hid_ref[...] = jnp.maximum(h, 0.0).astype(jnp.bfloat16).T


def _pool_hidden(gathered3, parity3, W_hidden, bh2):
    return pl.pallas_call(
        _pool_body,
        grid=(_BATCH // _BB,),
        in_specs=[
            pl.BlockSpec((_BB, _CTX, _GROW), lambda i: (i, 0, 0)),
            pl.BlockSpec((_BB, _CTX), lambda i: (i, 0)),
            pl.BlockSpec((_EMBED, _HIDDEN), lambda i: (0, 0)),
            pl.BlockSpec((1, _HIDDEN), lambda i: (0, 0)),
        ],
        out_specs=pl.BlockSpec((_HIDDEN, _BB), lambda i: (0, i)),
        out_shape=jax.ShapeDtypeStruct((_HIDDEN, _BATCH), jnp.bfloat16),
    )(gathered3, parity3, W_hidden, bh2)


_HB = _BATCH // 2


def _decode_step(t):
    u = t - _NV
    mid = jnp.logical_and(t >= _NV, t < 3 * _NV)
    phase = jnp.where(t < _NV, 0, jnp.where(mid, 1 - u % 2, 1))
    half = jnp.where(t < _NV, 0, jnp.where(mid, u % 2, 1))
    j = jnp.where(t < _NV, t, jnp.where(mid, u // 2, t - 3 * _NV))
    j_out = jnp.where(t < _NV, 0, jnp.where(mid, u // 2, t - 3 * _NV))
    half_out = jnp.where(t < 3 * _NV, 0, 1)
    return phase, half, j, j_out, half_out


def _tc_body(hid_ref, wo_ref, bo_ref, out_ref, m_scr, l_scr):
    t = pl.program_id(0)
    phase, half, j, _, _ = _decode_step(t)

    @pl.when(t == 0)
    def _init():
        m_scr[...] = jnp.full_like(m_scr, -jnp.inf)
        l_scr[...] = jnp.zeros_like(l_scr)

    logits = jnp.dot(wo_ref[...].astype(jnp.bfloat16), hid_ref[...],
                     preferred_element_type=jnp.float32) + bo_ref[...]

    @pl.when(phase == 0)
    def _accumulate():
        rows = jax.lax.broadcasted_iota(jnp.int32, (_VT, 1), 0) + j * _VT
        masked = jnp.where(rows < _VOCAB, logits, -jnp.inf)
        tile_max = jnp.max(masked, axis=0, keepdims=True)
        m_old = m_scr[pl.ds(half, 1), :]
        m_new = jnp.maximum(m_old, tile_max)
        l_scr[pl.ds(half, 1), :] = (
            l_scr[pl.ds(half, 1), :] * jnp.exp(m_old - m_new)
            + jnp.sum(jnp.exp(masked - m_new), axis=0, keepdims=True))
        m_scr[pl.ds(half, 1), :] = m_new

    @pl.when(phase == 1)
    def _write():
        out_ref[...] = logits - (m_scr[pl.ds(half, 1), :]
                                 + jnp.log(l_scr[pl.ds(half, 1), :]))


def _tc_forward(hidden_t, W_out_t, bo2):
    out_t = pl.pallas_call(
        _tc_body,
        grid=(4 * _NV,),
        in_specs=[
            pl.BlockSpec((_HIDDEN, _HB),
                         lambda t: (0, _decode_step(t)[1])),
            pl.BlockSpec((_VT, _HIDDEN),
                         lambda t: (_decode_step(t)[2], 0)),
            pl.BlockSpec((_VT, 1),
                         lambda t: (_decode_step(t)[2], 0)),
        ],
        out_specs=pl.BlockSpec(
            (_VT, _HB), lambda t: (_decode_step(t)[3], _decode_step(t)[4])),
        out_shape=jax.ShapeDtypeStruct((_VOCAB, _BATCH), jnp.float32),
        scratch_shapes=[
            pltpu.VMEM((2, _HB), jnp.float32),
            pltpu.VMEM((2, _HB), jnp.float32),
        ],
    )(hidden_t, W_out_t, bo2)
    return out_t


def kernel(x, emb_table, W_hidden, b_hidden, W_out, b_out):
    flat_idx = x.reshape(-1).astype(jnp.int32)
    table2 = emb_table.reshape(_VOCAB // 2, _GROW)
    gathered = _sc_gather(table2, flat_idx >> 1)
    gathered3 = gathered.reshape(_BATCH, _CTX, _GROW)
    parity3 = (flat_idx & 1).astype(jnp.float32).reshape(_BATCH, _CTX)
    bh2 = b_hidden.reshape(1, _HIDDEN)
    bo2 = b_out.reshape(_VOCAB, 1)
    hidden_t = _pool_hidden(gathered3, parity3, W_hidden, bh2)
    out_t = _tc_forward(hidden_t, W_out.T, bo2)
    return out_t.T


# speedup vs baseline: 1.3067x; 1.3067x over previous
import functools

import jax
import jax.numpy as jnp
from jax import lax
from jax.experimental import pallas as pl
from jax.experimental.pallas import tpu as pltpu
from jax.experimental.pallas import tpu_sc as plsc

_VOCAB = 100000
_EMBED = 64
_HIDDEN = 128
_BATCH = 1024
_CTX = 20

_NC = 2
_NS = 16
_NW = _NC * _NS
_NIDX = _BATCH * _CTX
_B_PER_W = _NIDX // _NW
_GROW = 2 * _EMBED

_BB = 128
_VT = 2048
_NV = (_VOCAB + _VT - 1) // _VT


def _sc_gather_kernel(table_hbm, idx_hbm, out_hbm, idx_v, rows_v, sem):
    wid = lax.axis_index("s") * _NC + lax.axis_index("c")
    base = wid * _B_PER_W
    pltpu.sync_copy(idx_hbm.at[pl.ds(base, _B_PER_W)], idx_v)
    pltpu.async_copy(table_hbm.at[idx_v], rows_v, sem).wait()
    pltpu.sync_copy(rows_v, out_hbm.at[pl.ds(base, _B_PER_W)])


def _sc_gather(emb_table, flat_idx):
    mesh = plsc.VectorSubcoreMesh(core_axis_name="c", subcore_axis_name="s")
    k = functools.partial(
        pl.kernel,
        mesh=mesh,
        out_type=jax.ShapeDtypeStruct((_NIDX, _GROW), jnp.float32),
        scratch_types=[
            pltpu.VMEM((_B_PER_W,), jnp.int32),
            pltpu.VMEM((_B_PER_W, _GROW), jnp.float32),
            pltpu.SemaphoreType.DMA,
        ],
    )(_sc_gather_kernel)
    return k(emb_table, flat_idx)


def _pool_body(gath_ref, par_ref, wh_ref, bh_ref, hid_ref):
    acc = jnp.zeros((_BB, _EMBED), jnp.float32)
    for c in range(_CTX):
        g0 = gath_ref[:, c, :_EMBED]
        g1 = gath_ref[:, c, _EMBED:]
        s = par_ref[:, c:c + 1]
        acc = acc + g0 * (1.0 - s) + g1 * s
    pooled = acc * (1.0 / _CTX)
    h = pooled @ wh_ref[...] + bh_ref[...]
    hid_ref[...] = jnp.maximum(h, 0.0).astype(jnp.bfloat16).T


def _pool_hidden(gathered3, parity3, W_hidden, bh2):
    return pl.pallas_call(
        _pool_body,
        grid=(_BATCH // _BB,),
        in_specs=[
            pl.BlockSpec((_BB, _CTX, _GROW), lambda i: (i, 0, 0)),
            pl.BlockSpec((_BB, _CTX), lambda i: (i, 0)),
            pl.BlockSpec((_EMBED, _HIDDEN), lambda i: (0, 0)),
            pl.BlockSpec((1, _HIDDEN), lambda i: (0, 0)),
        ],
        out_specs=pl.BlockSpec((_HIDDEN, _BB), lambda i: (0, i)),
        out_shape=jax.ShapeDtypeStruct((_HIDDEN, _BATCH), jnp.bfloat16),
    )(gathered3, parity3, W_hidden, bh2)


def _tc_body(hid_ref, wo_ref, bo_ref, out_ref, m_scr, l_scr):
    p = pl.program_id(0)
    j = pl.program_id(1)

    @pl.when((p == 0) & (j == 0))
    def _init():
        m_scr[...] = jnp.full_like(m_scr, -jnp.inf)
        l_scr[...] = jnp.zeros_like(l_scr)

    logits = jnp.dot(wo_ref[...].astype(jnp.bfloat16), hid_ref[...],
                     preferred_element_type=jnp.float32) + bo_ref[...]

    @pl.when(p == 0)
    def _accumulate():
        rows = jax.lax.broadcasted_iota(jnp.int32, (_VT, 1), 0) + j * _VT
        masked = jnp.where(rows < _VOCAB, logits, -jnp.inf)
        tile_max = jnp.max(masked, axis=0, keepdims=True)
        m_old = m_scr[...]
        m_new = jnp.maximum(m_old, tile_max)
        l_scr[...] = (l_scr[...] * jnp.exp(m_old - m_new)
                      + jnp.sum(jnp.exp(masked - m_new), axis=0, keepdims=True))
        m_scr[...] = m_new

    @pl.when(p == 1)
    def _write():
        out_ref[...] = logits - (m_scr[...] + jnp.log(l_scr[...]))


def _tc_forward(hidden_t, W_out_t, bo2):
    out_t = pl.pallas_call(
        _tc_body,
        grid=(2, _NV),
        in_specs=[
            pl.BlockSpec((_HIDDEN, _BATCH), lambda p, j: (0, 0)),
            pl.BlockSpec((_VT, _HIDDEN), lambda p, j: (j, 0)),
            pl.BlockSpec((_VT, 1), lambda p, j: (j, 0)),
        ],
        out_specs=pl.BlockSpec((_VT, _BATCH), lambda p, j: (p * j, 0)),
        out_shape=jax.ShapeDtypeStruct((_VOCAB, _BATCH), jnp.float32),
        scratch_shapes=[
            pltpu.VMEM((1, _BATCH), jnp.float32),
            pltpu.VMEM((1, _BATCH), jnp.float32),
        ],
    )(hidden_t, W_out_t, bo2)
    return out_t


def kernel(x, emb_table, W_hidden, b_hidden, W_out, b_out):
    flat_idx = x.reshape(-1).astype(jnp.int32)
    table2 = emb_table.reshape(_VOCAB // 2, _GROW)
    gathered = _sc_gather(table2, flat_idx >> 1)
    gathered3 = gathered.reshape(_BATCH, _CTX, _GROW)
    parity3 = (flat_idx & 1).astype(jnp.float32).reshape(_BATCH, _CTX)
    bh2 = b_hidden.reshape(1, _HIDDEN)
    bo2 = b_out.reshape(_VOCAB, 1)
    hidden_t = _pool_hidden(gathered3, parity3, W_hidden, bh2)
    out_t = _tc_forward(hidden_t, W_out.T, bo2)
    return out_t.T


# speedup vs baseline: 1.3179x; 1.0085x over previous
import functools

import jax
import jax.numpy as jnp
from jax import lax
from jax.experimental import pallas as pl
from jax.experimental.pallas import tpu as pltpu
from jax.experimental.pallas import tpu_sc as plsc

_VOCAB = 100000
_EMBED = 64
_HIDDEN = 128
_BATCH = 1024
_CTX = 20

_NC = 2
_NS = 16
_NW = _NC * _NS
_NIDX = _BATCH * _CTX
_B_PER_W = _NIDX // _NW
_GROW = 2 * _EMBED

_BB = 128
_VT = 3072
_NV = (_VOCAB + _VT - 1) // _VT


def _sc_gather_kernel(table_hbm, idx_hbm, out_hbm, idx_v, rows_v, sem):
    wid = lax.axis_index("s") * _NC + lax.axis_index("c")
    base = wid * _B_PER_W
    pltpu.sync_copy(idx_hbm.at[pl.ds(base, _B_PER_W)], idx_v)
    pltpu.async_copy(table_hbm.at[idx_v], rows_v, sem).wait()
    pltpu.sync_copy(rows_v, out_hbm.at[pl.ds(base, _B_PER_W)])


def _sc_gather(emb_table, flat_idx):
    mesh = plsc.VectorSubcoreMesh(core_axis_name="c", subcore_axis_name="s")
    k = functools.partial(
        pl.kernel,
        mesh=mesh,
        out_type=jax.ShapeDtypeStruct((_NIDX, _GROW), jnp.float32),
        scratch_types=[
            pltpu.VMEM((_B_PER_W,), jnp.int32),
            pltpu.VMEM((_B_PER_W, _GROW), jnp.float32),
            pltpu.SemaphoreType.DMA,
        ],
    )(_sc_gather_kernel)
    return k(emb_table, flat_idx)


def _pool_body(gath_ref, par_ref, wh_ref, bh_ref, hid_ref):
    acc = jnp.zeros((_BB, _EMBED), jnp.float32)
    for c in range(_CTX):
        g0 = gath_ref[:, c, :_EMBED]
        g1 = gath_ref[:, c, _EMBED:]
        s = par_ref[:, c:c + 1]
        acc = acc + g0 * (1.0 - s) + g1 * s
    pooled = acc * (1.0 / _CTX)
    h = pooled @ wh_ref[...] + bh_ref[...]
    hid_ref[...] = jnp.maximum(h, 0.0).astype(jnp.bfloat16).T


def _pool_hidden(gathered3, parity3, W_hidden, bh2):
    return pl.pallas_call(
        _pool_body,
        grid=(_BATCH // _BB,),
        in_specs=[
            pl.BlockSpec((_BB, _CTX, _GROW), lambda i: (i, 0, 0)),
            pl.BlockSpec((_BB, _CTX), lambda i: (i, 0)),
            pl.BlockSpec((_EMBED, _HIDDEN), lambda i: (0, 0)),
            pl.BlockSpec((1, _HIDDEN), lambda i: (0, 0)),
        ],
        out_specs=pl.BlockSpec((_HIDDEN, _BB), lambda i: (0, i)),
        out_shape=jax.ShapeDtypeStruct((_HIDDEN, _BATCH), jnp.bfloat16),
    )(gathered3, parity3, W_hidden, bh2)


def _tc_body(hid_ref, wo_ref, bo_ref, out_ref, m_scr, l_scr):
    p = pl.program_id(0)
    j = pl.program_id(1)

    @pl.when((p == 0) & (j == 0))
    def _init():
        m_scr[...] = jnp.full_like(m_scr, -jnp.inf)
        l_scr[...] = jnp.zeros_like(l_scr)

    logits = jnp.dot(wo_ref[...].astype(jnp.bfloat16), hid_ref[...],
                     preferred_element_type=jnp.float32) + bo_ref[...]

    @pl.when(p == 0)
    def _accumulate():
        rows = jax.lax.broadcasted_iota(jnp.int32, (_VT, 1), 0) + j * _VT
        masked = jnp.where(rows < _VOCAB, logits, -jnp.inf)
        tile_max = jnp.max(masked, axis=0, keepdims=True)
        m_old = m_scr[...]
        m_new = jnp.maximum(m_old, tile_max)
        l_scr[...] = (l_scr[...] * jnp.exp(m_old - m_new)
                      + jnp.sum(jnp.exp(masked - m_new), axis=0, keepdims=True))
        m_scr[...] = m_new

    @pl.when(p == 1)
    def _write():
        out_ref[...] = logits - (m_scr[...] + jnp.log(l_scr[...]))


def _tc_forward(hidden_t, W_out_t, bo2):
    out_t = pl.pallas_call(
        _tc_body,
        grid=(2, _NV),
        in_specs=[
            pl.BlockSpec((_HIDDEN, _BATCH), lambda p, j: (0, 0)),
            pl.BlockSpec((_VT, _HIDDEN), lambda p, j: (j, 0)),
            pl.BlockSpec((_VT, 1), lambda p, j: (j, 0)),
        ],
        out_specs=pl.BlockSpec((_VT, _BATCH), lambda p, j: (p * j, 0)),
        out_shape=jax.ShapeDtypeStruct((_VOCAB, _BATCH), jnp.float32),
        scratch_shapes=[
            pltpu.VMEM((1, _BATCH), jnp.float32),
            pltpu.VMEM((1, _BATCH), jnp.float32),
        ],
    )(hidden_t, W_out_t, bo2)
    return out_t


def kernel(x, emb_table, W_hidden, b_hidden, W_out, b_out):
    flat_idx = x.reshape(-1).astype(jnp.int32)
    table2 = emb_table.reshape(_VOCAB // 2, _GROW)
    gathered = _sc_gather(table2, flat_idx >> 1)
    gathered3 = gathered.reshape(_BATCH, _CTX, _GROW)
    parity3 = (flat_idx & 1).astype(jnp.float32).reshape(_BATCH, _CTX)
    bh2 = b_hidden.reshape(1, _HIDDEN)
    bo2 = b_out.reshape(_VOCAB, 1)
    hidden_t = _pool_hidden(gathered3, parity3, W_hidden, bh2)
    out_t = _tc_forward(hidden_t, W_out.T, bo2)
    return out_t.T


# speedup vs baseline: 1.4145x; 1.0733x over previous
import functools

import jax
import jax.numpy as jnp
from jax import lax
from jax.experimental import pallas as pl
from jax.experimental.pallas import tpu as pltpu
from jax.experimental.pallas import tpu_sc as plsc

_VOCAB = 100000
_EMBED = 64
_HIDDEN = 128
_BATCH = 1024
_CTX = 20

_NC = 2
_NS = 16
_NW = _NC * _NS
_NIDX = _BATCH * _CTX
_B_PER_W = _NIDX // _NW
_GROW = 2 * _EMBED

_BB = 128
_VT = 3072
_NV = (_VOCAB + _VT - 1) // _VT


def _sc_gather_kernel(table_hbm, idx_hbm, out_hbm, idx_v, rows_v, sem):
    wid = lax.axis_index("s") * _NC + lax.axis_index("c")
    base = wid * _B_PER_W
    pltpu.sync_copy(idx_hbm.at[pl.ds(base, _B_PER_W)], idx_v)
    pltpu.async_copy(table_hbm.at[idx_v], rows_v, sem).wait()
    pltpu.sync_copy(rows_v, out_hbm.at[pl.ds(base, _B_PER_W)])


def _sc_gather(emb_table, flat_idx):
    mesh = plsc.VectorSubcoreMesh(core_axis_name="c", subcore_axis_name="s")
    k = functools.partial(
        pl.kernel,
        mesh=mesh,
        out_type=jax.ShapeDtypeStruct((_NIDX, _GROW), jnp.float32),
        scratch_types=[
            pltpu.VMEM((_B_PER_W,), jnp.int32),
            pltpu.VMEM((_B_PER_W, _GROW), jnp.float32),
            pltpu.SemaphoreType.DMA,
        ],
    )(_sc_gather_kernel)
    return k(emb_table, flat_idx)


_RB = _BB * _CTX


def _pool_body(gath_ref, par_ref, m_ref, wh_ref, bh_ref, hid_ref):
    s = par_ref[...]
    lane = jax.lax.broadcasted_iota(jnp.int32, (_RB, _GROW), 1)
    w = jnp.where(lane < _EMBED, 1.0 - s, s)
    sel = gath_ref[...] * w
    pf = jnp.dot(m_ref[...], sel, preferred_element_type=jnp.float32)
    pooled = (pf[:, :_EMBED] + pf[:, _EMBED:]) * (1.0 / _CTX)
    h = pooled @ wh_ref[...] + bh_ref[...]
    hid_ref[...] = jnp.maximum(h, 0.0).astype(jnp.bfloat16).T


def _pool_hidden(gathered2, parity2, group_sel, W_hidden, bh2):
    return pl.pallas_call(
        _pool_body,
        grid=(_BATCH // _BB,),
        in_specs=[
            pl.BlockSpec((_RB, _GROW), lambda i: (i, 0)),
            pl.BlockSpec((_RB, 1), lambda i: (i, 0)),
            pl.BlockSpec((_BB, _RB), lambda i: (0, 0)),
            pl.BlockSpec((_EMBED, _HIDDEN), lambda i: (0, 0)),
            pl.BlockSpec((1, _HIDDEN), lambda i: (0, 0)),
        ],
        out_specs=pl.BlockSpec((_HIDDEN, _BB), lambda i: (0, i)),
        out_shape=jax.ShapeDtypeStruct((_HIDDEN, _BATCH), jnp.bfloat16),
    )(gathered2, parity2, group_sel, W_hidden, bh2)


def _tc_body(hid_ref, wo_ref, bo_ref, out_ref, m_scr, l_scr):
    p = pl.program_id(0)
    j = pl.program_id(1)

    @pl.when((p == 0) & (j == 0))
    def _init():
        m_scr[...] = jnp.full_like(m_scr, -jnp.inf)
        l_scr[...] = jnp.zeros_like(l_scr)

    logits = jnp.dot(wo_ref[...].astype(jnp.bfloat16), hid_ref[...],
                     preferred_element_type=jnp.float32) + bo_ref[...]

    @pl.when(p == 0)
    def _accumulate():
        rows = jax.lax.broadcasted_iota(jnp.int32, (_VT, 1), 0) + j * _VT
        masked = jnp.where(rows < _VOCAB, logits, -jnp.inf)
        tile_max = jnp.max(masked, axis=0, keepdims=True)
        m_old = m_scr[...]
        m_new = jnp.maximum(m_old, tile_max)
        l_scr[...] = (l_scr[...] * jnp.exp(m_old - m_new)
                      + jnp.sum(jnp.exp(masked - m_new), axis=0, keepdims=True))
        m_scr[...] = m_new

    @pl.when(p == 1)
    def _write():
        out_ref[...] = logits - (m_scr[...] + jnp.log(l_scr[...]))


def _tc_forward(hidden_t, W_out_t, bo2):
    out_t = pl.pallas_call(
        _tc_body,
        grid=(2, _NV),
        in_specs=[
            pl.BlockSpec((_HIDDEN, _BATCH), lambda p, j: (0, 0)),
            pl.BlockSpec((_VT, _HIDDEN), lambda p, j: (j, 0)),
            pl.BlockSpec((_VT, 1), lambda p, j: (j, 0)),
        ],
        out_specs=pl.BlockSpec((_VT, _BATCH), lambda p, j: (p * j, 0)),
        out_shape=jax.ShapeDtypeStruct((_VOCAB, _BATCH), jnp.float32),
        scratch_shapes=[
            pltpu.VMEM((1, _BATCH), jnp.float32),
            pltpu.VMEM((1, _BATCH), jnp.float32),
        ],
    )(hidden_t, W_out_t, bo2)
    return out_t


def kernel(x, emb_table, W_hidden, b_hidden, W_out, b_out):
    flat_idx = x.reshape(-1).astype(jnp.int32)
    table2 = emb_table.reshape(_VOCAB // 2, _GROW)
    gathered = _sc_gather(table2, flat_idx >> 1)
    parity2 = (flat_idx & 1).astype(jnp.float32).reshape(_NIDX, 1)
    group_sel = (jnp.arange(_RB, dtype=jnp.int32)[None, :] // _CTX
                 == jnp.arange(_BB, dtype=jnp.int32)[:, None]
                 ).astype(jnp.float32)
    bh2 = b_hidden.reshape(1, _HIDDEN)
    bo2 = b_out.reshape(_VOCAB, 1)
    hidden_t = _pool_hidden(gathered, parity2, group_sel, W_hidden, bh2)
    out_t = _tc_forward(hidden_t, W_out.T, bo2)
    return out_t.T


# speedup vs baseline: 1.4475x; 1.0234x over previous
import functools

import jax
import jax.numpy as jnp
from jax import lax
from jax.experimental import pallas as pl
from jax.experimental.pallas import tpu as pltpu
from jax.experimental.pallas import tpu_sc as plsc

_VOCAB = 100000
_EMBED = 64
_HIDDEN = 128
_BATCH = 1024
_CTX = 20

_NC = 2
_NS = 16
_NW = _NC * _NS
_NIDX = _BATCH * _CTX
_B_PER_W = _NIDX // _NW
_GROW = 2 * _EMBED

_BB = 128
_VT = 3072
_NV = (_VOCAB + _VT - 1) // _VT


def _sc_gather_kernel(table_hbm, idx_hbm, out_hbm, idx_v, rows_v, sem):
    wid = lax.axis_index("s") * _NC + lax.axis_index("c")
    base = wid * _B_PER_W
    pltpu.sync_copy(idx_hbm.at[pl.ds(base, _B_PER_W)], idx_v)
    pltpu.async_copy(table_hbm.at[idx_v], rows_v, sem).wait()
    pltpu.sync_copy(rows_v, out_hbm.at[pl.ds(base, _B_PER_W)])


def _sc_gather(emb_table, flat_idx):
    mesh = plsc.VectorSubcoreMesh(core_axis_name="c", subcore_axis_name="s")
    k = functools.partial(
        pl.kernel,
        mesh=mesh,
        out_type=jax.ShapeDtypeStruct((_NIDX, _GROW), jnp.float32),
        scratch_types=[
            pltpu.VMEM((_B_PER_W,), jnp.int32),
            pltpu.VMEM((_B_PER_W, _GROW), jnp.float32),
            pltpu.SemaphoreType.DMA,
        ],
    )(_sc_gather_kernel)
    return k(emb_table, flat_idx)


_RB = _BB * _CTX


def _pool_body(gath_ref, par_ref, m_ref, wh_ref, bh_ref, hid_ref):
    s = par_ref[...]
    lane = jax.lax.broadcasted_iota(jnp.int32, (_RB, _GROW), 1)
    w = jnp.where(lane < _EMBED, 1.0 - s, s)
    sel = gath_ref[...] * w
    pf = jnp.dot(m_ref[...], sel, preferred_element_type=jnp.float32)
    pooled = (pf[:, :_EMBED] + pf[:, _EMBED:]) * (1.0 / _CTX)
    h = pooled @ wh_ref[...] + bh_ref[...]
    hid_ref[...] = jnp.maximum(h, 0.0).astype(jnp.bfloat16).T


def _pool_hidden(gathered2, parity2, group_sel, W_hidden, bh2):
    return pl.pallas_call(
        _pool_body,
        grid=(_BATCH // _BB,),
        in_specs=[
            pl.BlockSpec((_RB, _GROW), lambda i: (i, 0)),
            pl.BlockSpec((_RB, 1), lambda i: (i, 0)),
            pl.BlockSpec((_BB, _RB), lambda i: (0, 0)),
            pl.BlockSpec((_EMBED, _HIDDEN), lambda i: (0, 0)),
            pl.BlockSpec((1, _HIDDEN), lambda i: (0, 0)),
        ],
        out_specs=pl.BlockSpec((_HIDDEN, _BB), lambda i: (0, i)),
        out_shape=jax.ShapeDtypeStruct((_HIDDEN, _BATCH), jnp.bfloat16),
    )(gathered2, parity2, group_sel, W_hidden, bh2)


def _tc_body(hid_ref, wo_ref, bo_ref, out_ref, m_scr, l_scr):
    p = pl.program_id(0)
    j = pl.program_id(1)

    @pl.when((p == 0) & (j == 0))
    def _init():
        m_scr[...] = jnp.full_like(m_scr, -jnp.inf)
        l_scr[...] = jnp.zeros_like(l_scr)

    logits = jnp.dot(wo_ref[...].astype(jnp.bfloat16), hid_ref[...],
                     preferred_element_type=jnp.float32) + bo_ref[...]

    @pl.when(p == 0)
    def _accumulate():
        rows = jax.lax.broadcasted_iota(jnp.int32, (_VT, 1), 0) + j * _VT
        masked = jnp.where(rows < _VOCAB, logits, -jnp.inf)
        tile_max = jnp.max(masked, axis=0, keepdims=True)
        m_old = m_scr[...]
        m_new = jnp.maximum(m_old, tile_max)
        e = jnp.exp(masked - m_new).astype(jnp.bfloat16)
        ones = jnp.ones((1, _VT), jnp.bfloat16)
        s = jnp.dot(ones, e, preferred_element_type=jnp.float32)
        l_scr[...] = l_scr[...] * jnp.exp(m_old - m_new) + s
        m_scr[...] = m_new

    @pl.when(p == 1)
    def _write():
        out_ref[...] = logits - (m_scr[...] + jnp.log(l_scr[...]))


def _tc_forward(hidden_t, W_out_t, bo2):
    out_t = pl.pallas_call(
        _tc_body,
        grid=(2, _NV),
        in_specs=[
            pl.BlockSpec((_HIDDEN, _BATCH), lambda p, j: (0, 0)),
            pl.BlockSpec((_VT, _HIDDEN), lambda p, j: (j, 0)),
            pl.BlockSpec((_VT, 1), lambda p, j: (j, 0)),
        ],
        out_specs=pl.BlockSpec((_VT, _BATCH), lambda p, j: (p * j, 0)),
        out_shape=jax.ShapeDtypeStruct((_VOCAB, _BATCH), jnp.float32),
        scratch_shapes=[
            pltpu.VMEM((1, _BATCH), jnp.float32),
            pltpu.VMEM((1, _BATCH), jnp.float32),
        ],
    )(hidden_t, W_out_t, bo2)
    return out_t


def kernel(x, emb_table, W_hidden, b_hidden, W_out, b_out):
    flat_idx = x.reshape(-1).astype(jnp.int32)
    table2 = emb_table.reshape(_VOCAB // 2, _GROW)
    gathered = _sc_gather(table2, flat_idx >> 1)
    parity2 = (flat_idx & 1).astype(jnp.float32).reshape(_NIDX, 1)
    group_sel = (jnp.arange(_RB, dtype=jnp.int32)[None, :] // _CTX
                 == jnp.arange(_BB, dtype=jnp.int32)[:, None]
                 ).astype(jnp.float32)
    bh2 = b_hidden.reshape(1, _HIDDEN)
    bo2 = b_out.reshape(_VOCAB, 1)
    hidden_t = _pool_hidden(gathered, parity2, group_sel, W_hidden, bh2)
    out_t = _tc_forward(hidden_t, W_out.T, bo2)
    return out_t.T
